# Initial kernel scaffold; baseline (speedup 1.0000x reference)
import jax
import jax.numpy as jnp
from jax.experimental import pallas as pl


def kernel(x, edge_index, h1, c1, h2, c2, W1, b1, U1, bU1, W2, bW2, U2, bU2):
    raise NotImplementedError("write your pallas kernel here")


# speedup vs baseline: 2.1942x; 2.1942x over previous
import functools

import jax
import jax.numpy as jnp
from jax import lax
from jax.experimental import pallas as pl
from jax.experimental.pallas import tpu as pltpu
from jax.experimental.pallas import tpu_sc as plsc

N = 10000
E = 320000
D = 128
NS = 16
NC = 2
CH = 160
EPT = CH * 128
E_PAD = NS * EPT
ROWS = E_PAD // 128
ACC_N = 10240
WB = ACC_N // NS


K = 16
NK = CH // K


def _sc_body(src4_hbm, dst_hbm, t_hbm, out_hbm,
             srcv, dstv, rows0, zbuf, acc, gsem):
    c = lax.axis_index("c")
    s = lax.axis_index("s")

    def zrow(i, carry):
        for jj in range(8):
            zbuf[i, pl.ds(jj * 16, 16)] = jnp.zeros((16,), jnp.float32)
        return carry
    lax.fori_loop(0, 64, zrow, 0)

    for p in range(2):
        a = 2 * p + c

        for q in range(WB // 64):
            pltpu.sync_copy(zbuf, acc.at[pl.ds(s * WB + q * 64, 64), :])
        plsc.subcore_barrier()

        def outer(kk, carry):
            pltpu.sync_copy(
                src4_hbm.at[pl.ds(a * ROWS + s * CH + kk * K, K)], srcv)
            pltpu.sync_copy(dst_hbm.at[pl.ds(s * CH + kk * K, K)], dstv)

            def step(j, carry2):
                pltpu.async_copy(t_hbm.at[srcv.at[j]], rows0, gsem).wait()
                pltpu.sync_copy(rows0, acc.at[dstv.at[j]], add=True)
                return carry2
            return lax.fori_loop(0, K, step, carry)
        lax.fori_loop(0, NK, outer, 0)
        plsc.subcore_barrier()

        pltpu.sync_copy(acc.at[pl.ds(s * WB, WB), :],
                        out_hbm.at[pl.ds(a * ACC_N + s * WB, WB), :])
        plsc.subcore_barrier()


_sc_call = pl.kernel(
    _sc_body,
    out_type=jax.ShapeDtypeStruct((4 * ACC_N, D), jnp.float32),
    mesh=plsc.VectorSubcoreMesh(core_axis_name="c", subcore_axis_name="s"),
    scratch_types=[
        pltpu.VMEM((K, 128), jnp.int32),
        pltpu.VMEM((K, 128), jnp.int32),
        pltpu.VMEM((128, D), jnp.float32),
        pltpu.VMEM((64, D), jnp.float32),
        pltpu.VMEM_SHARED((ACC_N, D), jnp.float32),
        pltpu.SemaphoreType.DMA,
    ],
)


def _tc_body(x_ref, ah1_ref, ac1_ref, ah2_ref, ac2_ref,
             w1_ref, u1_ref, w2_ref, u2_ref, b1_ref, b2_ref,
             h1_out, c1_out, h2_out, c2_out):
    f32 = jnp.float32
    g1 = (jnp.dot(x_ref[...], w1_ref[...], preferred_element_type=f32)
          + jnp.dot(ah1_ref[...], u1_ref[...], preferred_element_type=f32)
          + b1_ref[...])
    i1 = jax.nn.sigmoid(g1[:, 0:128])
    o1 = jax.nn.sigmoid(g1[:, 128:256])
    u1 = jnp.tanh(g1[:, 256:384])
    f1 = jax.nn.sigmoid(g1[:, 384:512])
    c1n = i1 * u1 + f1 * ac1_ref[...]
    g2 = (jnp.dot(c1n, w2_ref[...], preferred_element_type=f32)
          + jnp.dot(ah2_ref[...], u2_ref[...], preferred_element_type=f32)
          + b2_ref[...])
    i2 = jax.nn.sigmoid(g2[:, 0:128])
    o2 = jax.nn.sigmoid(g2[:, 128:256])
    u2 = jnp.tanh(g2[:, 256:384])
    f2 = jax.nn.sigmoid(g2[:, 384:512])
    c2n = i2 * u2 + f2 * ac2_ref[...]
    h1_out[...] = o1 * jnp.tanh(c1n)
    c1_out[...] = c1n
    h2_out[...] = o2 * jnp.tanh(c2n)
    c2_out[...] = c2n


def _dense(x, ah1, ac1, ah2, ac2, W1, U1, W2, U2, bias1, bias2):
    R = 1000
    grid = (N // R,)
    blk = lambda i: (i, 0)
    full = lambda i: (0, 0)
    return pl.pallas_call(
        _tc_body,
        grid=grid,
        in_specs=[
            pl.BlockSpec((R, D), blk),
            pl.BlockSpec((R, D), blk),
            pl.BlockSpec((R, D), blk),
            pl.BlockSpec((R, D), blk),
            pl.BlockSpec((R, D), blk),
            pl.BlockSpec((D, 4 * D), full),
            pl.BlockSpec((D, 4 * D), full),
            pl.BlockSpec((D, 4 * D), full),
            pl.BlockSpec((D, 4 * D), full),
            pl.BlockSpec((1, 4 * D), full),
            pl.BlockSpec((1, 4 * D), full),
        ],
        out_specs=[pl.BlockSpec((R, D), blk)] * 4,
        out_shape=[jax.ShapeDtypeStruct((N, D), jnp.float32)] * 4,
    )(x, ah1, ac1, ah2, ac2, W1, U1, W2, U2, bias1, bias2)


def kernel(x, edge_index, h1, c1, h2, c2, W1, b1, U1, bU1, W2, bW2, U2, bU2):
    src = edge_index[0]
    dst = edge_index[1]

    offs = (jnp.arange(4, dtype=jnp.int32) * N)[:, None]
    src4 = jnp.pad(src[None, :] + offs, ((0, 0), (0, E_PAD - E)),
                   constant_values=4 * N)
    src4 = src4.reshape(4 * ROWS, 128)
    dst_p = jnp.pad(dst, (0, E_PAD - E), constant_values=N).reshape(ROWS, 128)

    table = jnp.concatenate(
        [h1, c1, h2, c2, jnp.zeros((8, D), jnp.float32)], axis=0)

    ah_flat = _sc_call(src4, dst_p, table)
    ah = ah_flat.reshape(4, ACC_N, D)[:, :N, :]

    bias1 = (b1 + bU1).reshape(1, 4 * D)
    bias2 = (bW2 + bU2).reshape(1, 4 * D)
    h1n, c1n, h2n, c2n = _dense(x, ah[0], ah[1], ah[2], ah[3],
                                W1, U1, W2, U2, bias1, bias2)
    return jnp.stack([h1n, c1n, h2n, c2n])


# speedup vs baseline: 2.3318x; 1.0627x over previous
import functools

import jax
import jax.numpy as jnp
from jax import lax
from jax.experimental import pallas as pl
from jax.experimental.pallas import tpu as pltpu
from jax.experimental.pallas import tpu_sc as plsc

N = 10000
E = 320000
D = 128
NS = 16
NC = 2
CH = 160
EPT = CH * 128
E_PAD = NS * EPT
ROWS = E_PAD // 128
ACC_N = 10240
WB = ACC_N // NS


K = 16
NK = CH // K


def _sc_body(src4_hbm, dst_hbm, t_hbm, out_hbm,
             srcv, dstv, r0, r1, zbuf, acc, gs0, gs1, ss0, ss1):
    c = lax.axis_index("c")
    s = lax.axis_index("s")
    rbuf = (r0, r1)
    gsem = (gs0, gs1)
    ssem = (ss0, ss1)

    def zrow(i, carry):
        for jj in range(8):
            zbuf[i, pl.ds(jj * 16, 16)] = jnp.zeros((16,), jnp.float32)
        return carry
    lax.fori_loop(0, 64, zrow, 0)

    for p in range(2):
        a = 2 * p + c

        for q in range(WB // 64):
            pltpu.sync_copy(zbuf, acc.at[pl.ds(s * WB + q * 64, 64), :])
        plsc.subcore_barrier()

        def outer(kk, carry):
            pltpu.sync_copy(
                src4_hbm.at[pl.ds(a * ROWS + s * CH + kk * K, K)], srcv)
            pltpu.sync_copy(dst_hbm.at[pl.ds(s * CH + kk * K, K)], dstv)

            g = [None, None]
            sc = [None, None]
            g[0] = pltpu.make_async_copy(t_hbm.at[srcv.at[0]], r0, gs0)
            g[0].start()
            for j in range(K):
                b = j & 1
                g[b].wait()
                if j + 1 < K:
                    if j >= 1:
                        sc[1 - b].wait()
                    g[1 - b] = pltpu.make_async_copy(
                        t_hbm.at[srcv.at[j + 1]], rbuf[1 - b], gsem[1 - b])
                    g[1 - b].start()
                sc[b] = pltpu.async_copy(
                    rbuf[b], acc.at[dstv.at[j]], ssem[b], add=True)
            sc[0].wait()
            sc[1].wait()
            return carry
        lax.fori_loop(0, NK, outer, 0)
        plsc.subcore_barrier()

        pltpu.sync_copy(acc.at[pl.ds(s * WB, WB), :],
                        out_hbm.at[pl.ds(a * ACC_N + s * WB, WB), :])
        plsc.subcore_barrier()


_sc_call = pl.kernel(
    _sc_body,
    out_type=jax.ShapeDtypeStruct((4 * ACC_N, D), jnp.float32),
    mesh=plsc.VectorSubcoreMesh(core_axis_name="c", subcore_axis_name="s"),
    scratch_types=[
        pltpu.VMEM((K, 128), jnp.int32),
        pltpu.VMEM((K, 128), jnp.int32),
        pltpu.VMEM((128, D), jnp.float32),
        pltpu.VMEM((128, D), jnp.float32),
        pltpu.VMEM((64, D), jnp.float32),
        pltpu.VMEM_SHARED((ACC_N, D), jnp.float32),
        pltpu.SemaphoreType.DMA,
        pltpu.SemaphoreType.DMA,
        pltpu.SemaphoreType.DMA,
        pltpu.SemaphoreType.DMA,
    ],
)


def _tc_body(x_ref, ah1_ref, ac1_ref, ah2_ref, ac2_ref,
             w1_ref, u1_ref, w2_ref, u2_ref, b1_ref, b2_ref,
             h1_out, c1_out, h2_out, c2_out):
    f32 = jnp.float32
    g1 = (jnp.dot(x_ref[...], w1_ref[...], preferred_element_type=f32)
          + jnp.dot(ah1_ref[...], u1_ref[...], preferred_element_type=f32)
          + b1_ref[...])
    i1 = jax.nn.sigmoid(g1[:, 0:128])
    o1 = jax.nn.sigmoid(g1[:, 128:256])
    u1 = jnp.tanh(g1[:, 256:384])
    f1 = jax.nn.sigmoid(g1[:, 384:512])
    c1n = i1 * u1 + f1 * ac1_ref[...]
    g2 = (jnp.dot(c1n, w2_ref[...], preferred_element_type=f32)
          + jnp.dot(ah2_ref[...], u2_ref[...], preferred_element_type=f32)
          + b2_ref[...])
    i2 = jax.nn.sigmoid(g2[:, 0:128])
    o2 = jax.nn.sigmoid(g2[:, 128:256])
    u2 = jnp.tanh(g2[:, 256:384])
    f2 = jax.nn.sigmoid(g2[:, 384:512])
    c2n = i2 * u2 + f2 * ac2_ref[...]
    h1_out[...] = o1 * jnp.tanh(c1n)
    c1_out[...] = c1n
    h2_out[...] = o2 * jnp.tanh(c2n)
    c2_out[...] = c2n


def _dense(x, ah1, ac1, ah2, ac2, W1, U1, W2, U2, bias1, bias2):
    R = 1000
    grid = (N // R,)
    blk = lambda i: (i, 0)
    full = lambda i: (0, 0)
    return pl.pallas_call(
        _tc_body,
        grid=grid,
        in_specs=[
            pl.BlockSpec((R, D), blk),
            pl.BlockSpec((R, D), blk),
            pl.BlockSpec((R, D), blk),
            pl.BlockSpec((R, D), blk),
            pl.BlockSpec((R, D), blk),
            pl.BlockSpec((D, 4 * D), full),
            pl.BlockSpec((D, 4 * D), full),
            pl.BlockSpec((D, 4 * D), full),
            pl.BlockSpec((D, 4 * D), full),
            pl.BlockSpec((1, 4 * D), full),
            pl.BlockSpec((1, 4 * D), full),
        ],
        out_specs=[pl.BlockSpec((R, D), blk)] * 4,
        out_shape=[jax.ShapeDtypeStruct((N, D), jnp.float32)] * 4,
    )(x, ah1, ac1, ah2, ac2, W1, U1, W2, U2, bias1, bias2)


def kernel(x, edge_index, h1, c1, h2, c2, W1, b1, U1, bU1, W2, bW2, U2, bU2):
    src = edge_index[0]
    dst = edge_index[1]

    offs = (jnp.arange(4, dtype=jnp.int32) * N)[:, None]
    src4 = jnp.pad(src[None, :] + offs, ((0, 0), (0, E_PAD - E)),
                   constant_values=4 * N)
    src4 = src4.reshape(4 * ROWS, 128)
    dst_p = jnp.pad(dst, (0, E_PAD - E), constant_values=N).reshape(ROWS, 128)

    table = jnp.concatenate(
        [h1, c1, h2, c2, jnp.zeros((8, D), jnp.float32)], axis=0)

    ah_flat = _sc_call(src4, dst_p, table)
    ah = ah_flat.reshape(4, ACC_N, D)[:, :N, :]

    bias1 = (b1 + bU1).reshape(1, 4 * D)
    bias2 = (bW2 + bU2).reshape(1, 4 * D)
    h1n, c1n, h2n, c2n = _dense(x, ah[0], ah[1], ah[2], ah[3],
                                W1, U1, W2, U2, bias1, bias2)
    return jnp.stack([h1n, c1n, h2n, c2n])
